# baseline (speedup 1.0000x reference)
import functools

import jax
import jax.numpy as jnp
from jax import lax
from jax.experimental import pallas as pl
from jax.experimental.pallas import tpu as pltpu
from jax.experimental.pallas import tpu_sc as plsc

_D = 64
_NC = 2
_NS = 16
_NW = _NC * _NS


@functools.cache
def _gather_call(n_total: int, chunk: int):
    b_per_w = n_total // _NW
    n_chunks = b_per_w // chunk
    mesh = plsc.VectorSubcoreMesh(
        core_axis_name="c", subcore_axis_name="s",
        num_cores=_NC, num_subcores=_NS,
    )

    @functools.partial(
        pl.kernel,
        out_type=jax.ShapeDtypeStruct((n_total, _D), jnp.float32),
        mesh=mesh,
        scratch_types=[
            pltpu.VMEM((chunk,), jnp.int32),
            pltpu.VMEM((chunk, _D), jnp.float32),
            pltpu.SemaphoreType.DMA,
        ],
        compiler_params=pltpu.CompilerParams(use_tc_tiling_on_sc=False),
    )
    def body(idx_hbm, table_hbm, out_hbm, idx_v, rows_v, sem):
        wid = lax.axis_index("s") * _NC + lax.axis_index("c")
        base0 = wid * b_per_w

        def step(i, carry):
            base = base0 + i * chunk
            pltpu.sync_copy(idx_hbm.at[pl.ds(base, chunk)], idx_v)
            pltpu.async_copy(table_hbm.at[idx_v], rows_v, sem).wait()
            pltpu.sync_copy(rows_v, out_hbm.at[pl.ds(base, chunk)])
            return carry

        lax.fori_loop(0, n_chunks, step, 0)

    return body


def kernel(token_ids, weight):
    b, s = token_ids.shape
    flat = token_ids.reshape(b * s)
    out = _gather_call(b * s, 512)(flat, weight)
    return out.reshape(b, s, _D)


# speedup vs baseline: 1.0442x; 1.0442x over previous
import functools

import jax
import jax.numpy as jnp
from jax import lax
from jax.experimental import pallas as pl
from jax.experimental.pallas import tpu as pltpu
from jax.experimental.pallas import tpu_sc as plsc

_D = 64
_NC = 2
_NS = 16
_NW = _NC * _NS
_NBUF = 2


@functools.cache
def _gather_call(n_total: int, chunk: int):
    b_per_w = n_total // _NW
    n_chunks = b_per_w // chunk
    assert n_chunks % _NBUF == 0 and n_chunks >= 2 * _NBUF
    mesh = plsc.VectorSubcoreMesh(
        core_axis_name="c", subcore_axis_name="s",
        num_cores=_NC, num_subcores=_NS,
    )

    @functools.partial(
        pl.kernel,
        out_type=jax.ShapeDtypeStruct((n_total, _D), jnp.float32),
        mesh=mesh,
        scratch_types=[
            pltpu.VMEM((b_per_w,), jnp.int32),
            [pltpu.VMEM((chunk, _D), jnp.float32) for _ in range(_NBUF)],
            [pltpu.SemaphoreType.DMA for _ in range(_NBUF)],
            [pltpu.SemaphoreType.DMA for _ in range(_NBUF)],
        ],
        compiler_params=pltpu.CompilerParams(use_tc_tiling_on_sc=False),
    )
    def body(idx_hbm, table_hbm, out_hbm, idx_v, rows, gsem, wsem):
        wid = lax.axis_index("s") * _NC + lax.axis_index("c")
        base0 = wid * b_per_w
        pltpu.sync_copy(idx_hbm.at[pl.ds(base0, b_per_w)], idx_v)

        def start_gather(i, b):
            pltpu.async_copy(
                table_hbm.at[idx_v.at[pl.ds(i * chunk, chunk)]],
                rows[b], gsem[b])

        for b in range(_NBUF):
            start_gather(b, b)

        def outer(j, carry):
            for b in range(_NBUF):
                i = j * _NBUF + b
                pltpu.make_async_copy(
                    out_hbm.at[pl.ds(0, chunk)], rows[b], gsem[b]).wait()
                out_slice = out_hbm.at[pl.ds(base0 + i * chunk, chunk)]
                pltpu.async_copy(rows[b], out_slice, wsem[b])
                pltpu.make_async_copy(rows[b], out_slice, wsem[b]).wait()

                @pl.when(i + _NBUF < n_chunks)
                def _():
                    start_gather(i + _NBUF, b)
            return carry

        lax.fori_loop(0, n_chunks // _NBUF, outer, 0)

    return body


def kernel(token_ids, weight):
    b, s = token_ids.shape
    flat = token_ids.reshape(b * s)
    out = _gather_call(b * s, 800)(flat, weight)
    return out.reshape(b, s, _D)


# speedup vs baseline: 1.0676x; 1.0224x over previous
import functools

import jax
import jax.numpy as jnp
from jax import lax
from jax.experimental import pallas as pl
from jax.experimental.pallas import tpu as pltpu
from jax.experimental.pallas import tpu_sc as plsc

_D = 64
_NC = 2
_NS = 16
_NW = _NC * _NS
_NBUF = 2


@functools.cache
def _gather_call(n_total: int, chunk: int):
    b_per_w = n_total // _NW
    n_chunks = b_per_w // chunk
    assert n_chunks % _NBUF == 0 and n_chunks >= 2 * _NBUF
    mesh = plsc.VectorSubcoreMesh(
        core_axis_name="c", subcore_axis_name="s",
        num_cores=_NC, num_subcores=_NS,
    )

    @functools.partial(
        pl.kernel,
        out_type=jax.ShapeDtypeStruct((n_total, _D), jnp.float32),
        mesh=mesh,
        scratch_types=[
            pltpu.VMEM((b_per_w,), jnp.int32),
            [pltpu.VMEM((chunk, _D), jnp.float32) for _ in range(_NBUF)],
            [pltpu.SemaphoreType.DMA for _ in range(_NBUF)],
            [pltpu.SemaphoreType.DMA for _ in range(_NBUF)],
        ],
        compiler_params=pltpu.CompilerParams(use_tc_tiling_on_sc=False),
    )
    def body(idx_hbm, table_hbm, out_hbm, idx_v, rows, gsem, wsem):
        wid = lax.axis_index("s") * _NC + lax.axis_index("c")
        base0 = wid * b_per_w
        pltpu.sync_copy(idx_hbm.at[pl.ds(base0, b_per_w)], idx_v)

        def start_gather(i, b):
            pltpu.async_copy(
                table_hbm.at[idx_v.at[pl.ds(i * chunk, chunk)]],
                rows[b], gsem[b])

        for b in range(_NBUF):
            start_gather(b, b)

        def outer(j, carry):
            for b in range(_NBUF):
                i = j * _NBUF + b
                pltpu.make_async_copy(
                    out_hbm.at[pl.ds(0, chunk)], rows[b], gsem[b]).wait()
                out_slice = out_hbm.at[pl.ds(base0 + i * chunk, chunk)]
                pltpu.async_copy(rows[b], out_slice, wsem[b])
                pltpu.make_async_copy(rows[b], out_slice, wsem[b]).wait()

                @pl.when(i + _NBUF < n_chunks)
                def _():
                    start_gather(i + _NBUF, b)
            return carry

        lax.fori_loop(0, n_chunks // _NBUF, outer, 0)

    return body


def kernel(token_ids, weight):
    b, s = token_ids.shape
    flat = token_ids.T.reshape(b * s)
    out = _gather_call(b * s, 800)(flat, weight)
    return out.reshape(s, b, _D).transpose(1, 0, 2)
